# baseline (speedup 1.0000x reference)
import functools

import jax
import jax.numpy as jnp
from jax import lax
from jax.experimental import pallas as pl
from jax.experimental.pallas import tpu as pltpu
from jax.experimental.pallas import tpu_sc as plsc

N = 10000
NP = 10240
E = 320000
D = 128
DOUT = 16
EPS = 1e-5

NC = 2
NS = 16
NW = NC * NS
EPT = E // NW
K = 40
GRP = 5
NGRP = EPT // (K * GRP)
KW = 80
RPT = NP // NS

_mesh = plsc.VectorSubcoreMesh(
    core_axis_name="c", subcore_axis_name="s", num_cores=NC, num_subcores=NS
)


def _zero_rows(buf, nrows, ncols):
    def body(i, _):
        for k in range(ncols // 16):
            buf[i, pl.ds(k * 16, 16)] = jnp.zeros((16,), jnp.float32)
        return 0

    lax.fori_loop(0, nrows, body, 0)


def _slab_writeout(acc_sh, out_hbm, c, base0, rows_v):
    for j in range(RPT // KW):
        pltpu.sync_copy(acc_sh.at[pl.ds(base0 + j * KW, KW)], rows_v)
        pltpu.sync_copy(rows_v, out_hbm.at[c, pl.ds(base0 + j * KW, KW)])


@functools.partial(
    pl.kernel,
    out_type=jax.ShapeDtypeStruct((NC, NP, D), jnp.float32),
    mesh=_mesh,
    scratch_types=[
        pltpu.VMEM((KW, D), jnp.float32),
        pltpu.VMEM((GRP, K), jnp.int32),
        pltpu.VMEM_SHARED((NP, D), jnp.float32),
        pltpu.SemaphoreType.DMA((GRP,)),
    ],
)
def _deg(dst_hbm, out_hbm, ones_v, didx, acc_deg_sh, sems):
    c = lax.axis_index("c")
    s = lax.axis_index("s")
    wid = s * NC + c
    base0 = s * RPT

    _zero_rows(ones_v, KW, D)
    for j in range(RPT // KW):
        pltpu.sync_copy(ones_v, acc_deg_sh.at[pl.ds(base0 + j * KW, KW)])
    plsc.subcore_barrier()

    def fill(i, _):
        for k in range(D // 16):
            ones_v[i, pl.ds(k * 16, 16)] = jnp.ones((16,), jnp.float32)
        return 0

    lax.fori_loop(0, KW, fill, 0)

    def body(i, _):
        for k in range(GRP):
            base = wid * EPT + (i * GRP + k) * K

            @pl.when(i > 0)
            def _():
                pltpu.make_async_copy(
                    ones_v.at[pl.ds(0, K), :],
                    acc_deg_sh.at[didx.at[k]],
                    sems.at[k],
                ).wait()

            pltpu.sync_copy(dst_hbm.at[pl.ds(base, K)], didx.at[k])
            pltpu.async_copy(
                ones_v.at[pl.ds(0, K), :], acc_deg_sh.at[didx.at[k]],
                sems.at[k], add=True,
            )
        return 0

    lax.fori_loop(0, NGRP, body, 0)
    for k in range(GRP):
        pltpu.make_async_copy(
            ones_v.at[pl.ds(0, K), :], acc_deg_sh.at[didx.at[k]], sems.at[k]
        ).wait()

    plsc.subcore_barrier()
    _slab_writeout(acc_deg_sh, out_hbm, c, base0, ones_v)


@functools.partial(
    pl.kernel,
    out_type=jax.ShapeDtypeStruct((NC, NP, D), jnp.float32),
    mesh=_mesh,
    scratch_types=[
        pltpu.VMEM((KW, D), jnp.float32),
        pltpu.VMEM((GRP, K, D), jnp.float32),
        pltpu.VMEM((GRP, K), jnp.int32),
        pltpu.VMEM((GRP, K), jnp.int32),
        pltpu.VMEM_SHARED((NP, D), jnp.float32),
        pltpu.SemaphoreType.DMA((GRP,)),
        pltpu.SemaphoreType.DMA((GRP,)),
    ],
)
def _agg(g_hbm, src_hbm, dst_hbm, out_hbm, stage_v, rows_v, sidx, didx, acc_sh, semg, sems):
    c = lax.axis_index("c")
    s = lax.axis_index("s")
    wid = s * NC + c
    base0 = s * RPT

    _zero_rows(stage_v, KW, D)
    for j in range(RPT // KW):
        pltpu.sync_copy(stage_v, acc_sh.at[pl.ds(base0 + j * KW, KW)])
    plsc.subcore_barrier()

    def body(i, _):
        for k in range(GRP):
            base = wid * EPT + (i * GRP + k) * K

            @pl.when(i > 0)
            def _():
                pltpu.make_async_copy(
                    rows_v.at[k], acc_sh.at[didx.at[k]], sems.at[k]
                ).wait()

            pltpu.sync_copy(src_hbm.at[pl.ds(base, K)], sidx.at[k])
            pltpu.sync_copy(dst_hbm.at[pl.ds(base, K)], didx.at[k])
            pltpu.async_copy(g_hbm.at[sidx.at[k]], rows_v.at[k], semg.at[k])
        for k in range(GRP):
            pltpu.make_async_copy(
                g_hbm.at[sidx.at[k]], rows_v.at[k], semg.at[k]
            ).wait()
            pltpu.async_copy(
                rows_v.at[k], acc_sh.at[didx.at[k]], sems.at[k], add=True
            )
        return 0

    lax.fori_loop(0, NGRP, body, 0)
    for k in range(GRP):
        pltpu.make_async_copy(
            rows_v.at[k], acc_sh.at[didx.at[k]], sems.at[k]
        ).wait()

    plsc.subcore_barrier()
    _slab_writeout(acc_sh, out_hbm, c, base0, stage_v)


def _tc1_body(x_ref, w_ref, degs_ref, g_ref, dinv_ref):
    deg = degs_ref[0][0:N, 0:1] + degs_ref[1][0:N, 0:1] + 1.0
    dinv = lax.rsqrt(deg)
    dinv_ref[...] = dinv
    g_ref[...] = (
        jnp.dot(x_ref[...], w_ref[...], preferred_element_type=jnp.float32) * dinv
    )


def _tc_mid_body(acc_ref, g_ref, dinv_ref, b_ref, gamma_ref, beta_ref, w_ref, out_ref):
    dinv = dinv_ref[...]
    sacc = (acc_ref[0][0:N] + acc_ref[1][0:N] + g_ref[...]) * dinv + b_ref[...]
    h = jnp.maximum(sacc, 0.0)
    mean = jnp.mean(h, axis=0, keepdims=True)
    var = jnp.mean((h - mean) ** 2, axis=0, keepdims=True)
    hn = (h - mean) * lax.rsqrt(var + EPS) * gamma_ref[...] + beta_ref[...]
    out_ref[...] = (
        jnp.dot(hn, w_ref[...], preferred_element_type=jnp.float32) * dinv
    )


def _tc_fin_body(acc_ref, g_ref, dinv_ref, b_ref, wc_ref, bc_ref, out_ref):
    sacc = (acc_ref[0][0:N] + acc_ref[1][0:N] + g_ref[...]) * dinv_ref[...] + b_ref[...]
    h = jnp.maximum(sacc, 0.0)
    pooled = jnp.mean(h, axis=0, keepdims=True)
    out_ref[...] = (
        jnp.dot(pooled, wc_ref[...], preferred_element_type=jnp.float32) + bc_ref[...]
    )


_tc1 = pl.pallas_call(
    _tc1_body,
    out_shape=(
        jax.ShapeDtypeStruct((N, D), jnp.float32),
        jax.ShapeDtypeStruct((N, 1), jnp.float32),
    ),
)

_tc_mid = pl.pallas_call(
    _tc_mid_body,
    out_shape=jax.ShapeDtypeStruct((N, D), jnp.float32),
)

_tc_fin = pl.pallas_call(
    _tc_fin_body,
    out_shape=jax.ShapeDtypeStruct((1, DOUT), jnp.float32),
)


def kernel(x, edge_index, W1, b1, W2, b2, W3, b3, gamma, beta, Wc, bc):
    src = edge_index[0].astype(jnp.int32)
    dst = edge_index[1].astype(jnp.int32)
    b1r = b1.reshape(1, D)
    b2r = b2.reshape(1, D)
    b3r = b3.reshape(1, D)
    gr = gamma.reshape(1, D)
    br = beta.reshape(1, D)
    bcr = bc.reshape(1, DOUT)

    degs = _deg(dst)
    g1, dinv = _tc1(x, W1, degs)
    acc1 = _agg(g1, src, dst)
    g2 = _tc_mid(acc1, g1, dinv, b1r, gr, br, W2)
    acc2 = _agg(g2, src, dst)
    g3 = _tc_mid(acc2, g2, dinv, b2r, gr, br, W3)
    acc3 = _agg(g3, src, dst)
    return _tc_fin(acc3, g3, dinv, b3r, Wc, bcr)


# speedup vs baseline: 1.9135x; 1.9135x over previous
import functools

import jax
import jax.numpy as jnp
from jax import lax
from jax.experimental import pallas as pl
from jax.experimental.pallas import tpu as pltpu
from jax.experimental.pallas import tpu_sc as plsc

N = 10000
NP = 10240
E = 320000
D = 128
DOUT = 16
EPS = 1e-5

NC = 2
NS = 16
NW = NC * NS
EPT = E // NW
K = 128
NCH = 80
EPTP = NCH * K
EP = EPTP * NW
KW = 128
RPT = NP // NS

_mesh = plsc.VectorSubcoreMesh(
    core_axis_name="c", subcore_axis_name="s", num_cores=NC, num_subcores=NS
)


def _zero_slot(buf):
    def body(i, _):
        for k in range(D // 16):
            buf[0, i, pl.ds(k * 16, 16)] = jnp.zeros((16,), jnp.float32)
        return 0

    lax.fori_loop(0, K, body, 0)


def _zero_acc(acc_sh, base0, rows_v):
    _zero_slot(rows_v)
    for j in range(RPT // KW):
        pltpu.sync_copy(rows_v.at[0], acc_sh.at[pl.ds(base0 + j * KW, KW)])


def _slab_writeout(acc_sh, out_hbm, c, base0, rows_v, sems):
    for j in range(RPT // KW):
        k = j % 2
        if j >= 2:
            pltpu.make_async_copy(
                rows_v.at[k], out_hbm.at[c, pl.ds(base0, KW)], sems.at[k]
            ).wait()
        pltpu.sync_copy(acc_sh.at[pl.ds(base0 + j * KW, KW)], rows_v.at[k])
        pltpu.async_copy(
            rows_v.at[k], out_hbm.at[c, pl.ds(base0 + j * KW, KW)], sems.at[k]
        )
    for k in range(2):
        pltpu.make_async_copy(
            rows_v.at[k], out_hbm.at[c, pl.ds(base0, KW)], sems.at[k]
        ).wait()


@functools.partial(
    pl.kernel,
    out_type=jax.ShapeDtypeStruct((NC, NP, D), jnp.float32),
    mesh=_mesh,
    scratch_types=[
        pltpu.VMEM((2, K, D), jnp.float32),
        pltpu.VMEM((8, K), jnp.int32),
        pltpu.VMEM_SHARED((NP, D), jnp.float32),
        pltpu.SemaphoreType.DMA((4,)),
        pltpu.SemaphoreType.DMA((8,)),
    ],
)
def _deg(ei_hbm, out_hbm, rows_v, didx, acc_sh, sems, semi):
    c = lax.axis_index("c")
    s = lax.axis_index("s")
    wid = s * NC + c
    base0 = s * RPT

    _zero_acc(acc_sh, base0, rows_v)

    def fill(i, _):
        for k in range(D // 16):
            rows_v[0, i, pl.ds(k * 16, 16)] = jnp.ones((16,), jnp.float32)
        return 0

    lax.fori_loop(0, K, fill, 0)
    plsc.subcore_barrier()

    for q in range(4):
        pltpu.async_copy(ei_hbm.at[wid, 1, q], didx.at[q], semi.at[q])

    def _scatter(cc, q8, q4):
        pltpu.make_async_copy(
            ei_hbm.at[wid, 1, cc], didx.at[q8], semi.at[q8]
        ).wait()
        pltpu.async_copy(
            rows_v.at[0], acc_sh.at[didx.at[q8]], sems.at[q4], add=True
        )

    for u in range(4):
        pltpu.async_copy(ei_hbm.at[wid, 1, u + 4], didx.at[u + 4], semi.at[u + 4])
        _scatter(u, u, u)

    def body(i, _):
        for u in range(8):
            cc = 4 + i * 8 + u
            q8 = (4 + u) % 8
            q4 = u % 4
            pltpu.make_async_copy(
                rows_v.at[0], acc_sh.at[didx.at[q8]], sems.at[q4]
            ).wait()
            pltpu.async_copy(
                ei_hbm.at[wid, 1, cc + 4], didx.at[(cc + 4) % 8], semi.at[(cc + 4) % 8]
            )
            _scatter(cc, q8, q4)
        return 0

    lax.fori_loop(0, (NCH - 8) // 8, body, 0)
    for u in range(4):
        cc = NCH - 4 + u
        q8 = cc % 8
        q4 = cc % 4
        pltpu.make_async_copy(
            rows_v.at[0], acc_sh.at[didx.at[q8]], sems.at[q4]
        ).wait()
        _scatter(cc, q8, q4)
    for q4 in range(4):
        pltpu.make_async_copy(
            rows_v.at[0], acc_sh.at[didx.at[0]], sems.at[q4]
        ).wait()

    plsc.subcore_barrier()
    _slab_writeout(acc_sh, out_hbm, c, base0, rows_v, sems)


@functools.partial(
    pl.kernel,
    out_type=jax.ShapeDtypeStruct((NC, NP, D), jnp.float32),
    mesh=_mesh,
    scratch_types=[
        pltpu.VMEM((2, K, D), jnp.float32),
        pltpu.VMEM((4, K), jnp.int32),
        pltpu.VMEM((4, K), jnp.int32),
        pltpu.VMEM_SHARED((NP, D), jnp.float32),
        pltpu.SemaphoreType.DMA((2,)),
        pltpu.SemaphoreType.DMA((2,)),
        pltpu.SemaphoreType.DMA((4,)),
        pltpu.SemaphoreType.DMA((4,)),
    ],
)
def _agg(g_hbm, ei_hbm, out_hbm, rows_v, sidx, didx, acc_sh, semg, sems, semis, semid):
    c = lax.axis_index("c")
    s = lax.axis_index("s")
    wid = s * NC + c
    base0 = s * RPT

    _zero_acc(acc_sh, base0, rows_v)
    plsc.subcore_barrier()

    def _load_idx(cc, q):
        pltpu.async_copy(ei_hbm.at[wid, 0, cc], sidx.at[q], semis.at[q])
        pltpu.async_copy(ei_hbm.at[wid, 1, cc], didx.at[q], semid.at[q])

    def _fire_gather(cc, q, p):
        pltpu.make_async_copy(
            ei_hbm.at[wid, 0, cc], sidx.at[q], semis.at[q]
        ).wait()
        pltpu.make_async_copy(
            ei_hbm.at[wid, 1, cc], didx.at[q], semid.at[q]
        ).wait()
        pltpu.async_copy(g_hbm.at[sidx.at[q]], rows_v.at[p], semg.at[p])

    _load_idx(0, 0)
    _load_idx(1, 1)
    _fire_gather(0, 0, 0)
    pltpu.make_async_copy(g_hbm.at[sidx.at[0]], rows_v.at[0], semg.at[0]).wait()
    pltpu.async_copy(rows_v.at[0], acc_sh.at[didx.at[0]], sems.at[0], add=True)
    _load_idx(2, 2)
    _fire_gather(1, 1, 1)
    pltpu.make_async_copy(g_hbm.at[sidx.at[1]], rows_v.at[1], semg.at[1]).wait()
    pltpu.async_copy(rows_v.at[1], acc_sh.at[didx.at[1]], sems.at[1], add=True)
    _load_idx(3, 3)
    pltpu.make_async_copy(rows_v.at[0], acc_sh.at[didx.at[0]], sems.at[0]).wait()
    _fire_gather(2, 2, 0)

    def body(i, _):
        for u in range(4):
            cc = 2 + i * 4 + u
            q = (2 + u) % 4
            p = u % 2
            pltpu.make_async_copy(
                g_hbm.at[sidx.at[q]], rows_v.at[p], semg.at[p]
            ).wait()
            pltpu.async_copy(
                rows_v.at[p], acc_sh.at[didx.at[q]], sems.at[p], add=True
            )
            _load_idx(cc + 2, (q + 2) % 4)
            pltpu.make_async_copy(
                rows_v.at[1 - p], acc_sh.at[didx.at[(q + 3) % 4]], sems.at[1 - p]
            ).wait()
            _fire_gather(cc + 1, (q + 1) % 4, 1 - p)
        return 0

    lax.fori_loop(0, (NCH - 4) // 4, body, 0)

    pltpu.make_async_copy(g_hbm.at[sidx.at[2]], rows_v.at[0], semg.at[0]).wait()
    pltpu.async_copy(rows_v.at[0], acc_sh.at[didx.at[2]], sems.at[0], add=True)
    pltpu.make_async_copy(rows_v.at[1], acc_sh.at[didx.at[1]], sems.at[1]).wait()
    _fire_gather(79, 3, 1)
    pltpu.make_async_copy(g_hbm.at[sidx.at[3]], rows_v.at[1], semg.at[1]).wait()
    pltpu.async_copy(rows_v.at[1], acc_sh.at[didx.at[3]], sems.at[1], add=True)
    pltpu.make_async_copy(rows_v.at[0], acc_sh.at[didx.at[2]], sems.at[0]).wait()
    pltpu.make_async_copy(rows_v.at[1], acc_sh.at[didx.at[3]], sems.at[1]).wait()

    plsc.subcore_barrier()
    _slab_writeout(acc_sh, out_hbm, c, base0, rows_v, sems)


def _tc1_body(x_ref, w_ref, degs_ref, g_ref, dinv_ref):
    deg = degs_ref[0][0:N, 0:1] + degs_ref[1][0:N, 0:1] + 1.0
    dinv = lax.rsqrt(deg)
    dinv_ref[...] = dinv
    g_ref[...] = (
        jnp.dot(x_ref[...], w_ref[...], preferred_element_type=jnp.float32) * dinv
    )


def _tc_mid_body(acc_ref, g_ref, dinv_ref, b_ref, gamma_ref, beta_ref, w_ref, out_ref):
    dinv = dinv_ref[...]
    sacc = (acc_ref[0][0:N] + acc_ref[1][0:N] + g_ref[...]) * dinv + b_ref[...]
    h = jnp.maximum(sacc, 0.0)
    mean = jnp.mean(h, axis=0, keepdims=True)
    var = jnp.mean((h - mean) ** 2, axis=0, keepdims=True)
    hn = (h - mean) * lax.rsqrt(var + EPS) * gamma_ref[...] + beta_ref[...]
    out_ref[...] = (
        jnp.dot(hn, w_ref[...], preferred_element_type=jnp.float32) * dinv
    )


def _tc_fin_body(acc_ref, g_ref, dinv_ref, b_ref, wc_ref, bc_ref, out_ref):
    sacc = (acc_ref[0][0:N] + acc_ref[1][0:N] + g_ref[...]) * dinv_ref[...] + b_ref[...]
    h = jnp.maximum(sacc, 0.0)
    pooled = jnp.mean(h, axis=0, keepdims=True)
    out_ref[...] = (
        jnp.dot(pooled, wc_ref[...], preferred_element_type=jnp.float32) + bc_ref[...]
    )


_tc1 = pl.pallas_call(
    _tc1_body,
    out_shape=(
        jax.ShapeDtypeStruct((N, D), jnp.float32),
        jax.ShapeDtypeStruct((N, 1), jnp.float32),
    ),
)

_tc_mid = pl.pallas_call(
    _tc_mid_body,
    out_shape=jax.ShapeDtypeStruct((N, D), jnp.float32),
)

_tc_fin = pl.pallas_call(
    _tc_fin_body,
    out_shape=jax.ShapeDtypeStruct((1, DOUT), jnp.float32),
)


def kernel(x, edge_index, W1, b1, W2, b2, W3, b3, gamma, beta, Wc, bc):
    npad = EP - E
    ei32 = edge_index.astype(jnp.int32)
    pad_src = (jnp.arange(npad, dtype=jnp.int32) * 131) % N
    pad_dst = N + (jnp.arange(npad, dtype=jnp.int32) % (NP - N))
    srcf = jnp.concatenate([ei32[0], pad_src])
    dstf = jnp.concatenate([ei32[1], pad_dst])
    ei = (
        jnp.stack([srcf, dstf])
        .reshape(2, NW, NCH, K)
        .transpose(1, 0, 2, 3)
    )
    b1r = b1.reshape(1, D)
    b2r = b2.reshape(1, D)
    b3r = b3.reshape(1, D)
    gr = gamma.reshape(1, D)
    br = beta.reshape(1, D)
    bcr = bc.reshape(1, DOUT)

    degs = _deg(ei)
    g1, dinv = _tc1(x, W1, degs)
    acc1 = _agg(g1, ei)
    g2 = _tc_mid(acc1, g1, dinv, b1r, gr, br, W2)
    acc2 = _agg(g2, ei)
    g3 = _tc_mid(acc2, g2, dinv, b2r, gr, br, W3)
    acc3 = _agg(g3, ei)
    return _tc_fin(acc3, g3, dinv, b3r, Wc, bcr)


# speedup vs baseline: 2.1891x; 1.1441x over previous
import functools

import jax
import jax.numpy as jnp
from jax import lax
from jax.experimental import pallas as pl
from jax.experimental.pallas import tpu as pltpu
from jax.experimental.pallas import tpu_sc as plsc

N = 10000
NP = 10240
E = 320000
D = 128
DOUT = 16
EPS = 1e-5

NC = 2
NS = 16
NW = NC * NS
EPT = E // NW
K = 128
NCH = 80
EPTP = NCH * K
EP = EPTP * NW
KW = 128
RPT = NP // NS

_mesh = plsc.VectorSubcoreMesh(
    core_axis_name="c", subcore_axis_name="s", num_cores=NC, num_subcores=NS
)


def _zero_slot(buf):
    def body(i, _):
        for k in range(D // 16):
            buf[0, i, pl.ds(k * 16, 16)] = jnp.zeros((16,), jnp.float32)
        return 0

    lax.fori_loop(0, K, body, 0)


def _zero_acc(acc_sh, base0, rows_v):
    _zero_slot(rows_v)
    for j in range(RPT // KW):
        pltpu.sync_copy(rows_v.at[0], acc_sh.at[pl.ds(base0 + j * KW, KW)])


def _slab_writeout(acc_sh, out_hbm, c, base0, rows_v, sems):
    for j in range(RPT // KW):
        k = j % 2
        if j >= 2:
            pltpu.make_async_copy(
                rows_v.at[k], out_hbm.at[c, pl.ds(base0, KW)], sems.at[k]
            ).wait()
        pltpu.sync_copy(acc_sh.at[pl.ds(base0 + j * KW, KW)], rows_v.at[k])
        pltpu.async_copy(
            rows_v.at[k], out_hbm.at[c, pl.ds(base0 + j * KW, KW)], sems.at[k]
        )
    for k in range(2):
        pltpu.make_async_copy(
            rows_v.at[k], out_hbm.at[c, pl.ds(base0, KW)], sems.at[k]
        ).wait()


@functools.partial(
    pl.kernel,
    out_type=jax.ShapeDtypeStruct((NC, NP, D), jnp.float32),
    mesh=_mesh,
    scratch_types=[
        pltpu.VMEM((2, K, D), jnp.float32),
        pltpu.VMEM((8, K), jnp.int32),
        pltpu.VMEM_SHARED((NP, D), jnp.float32),
        pltpu.SemaphoreType.DMA((4,)),
        pltpu.SemaphoreType.DMA((8,)),
    ],
)
def _deg(ei_hbm, out_hbm, rows_v, didx, acc_sh, sems, semi):
    c = lax.axis_index("c")
    s = lax.axis_index("s")
    wid = s * NC + c
    base0 = s * RPT

    _zero_acc(acc_sh, base0, rows_v)

    def fill(i, _):
        for k in range(D // 16):
            rows_v[0, i, pl.ds(k * 16, 16)] = jnp.ones((16,), jnp.float32)
        return 0

    lax.fori_loop(0, K, fill, 0)
    plsc.subcore_barrier()

    for q in range(4):
        pltpu.async_copy(ei_hbm.at[wid, 1, q], didx.at[q], semi.at[q])

    def _scatter(cc, q8, q4):
        pltpu.make_async_copy(
            ei_hbm.at[wid, 1, cc], didx.at[q8], semi.at[q8]
        ).wait()
        pltpu.async_copy(
            rows_v.at[0], acc_sh.at[didx.at[q8]], sems.at[q4], add=True
        )

    for u in range(4):
        pltpu.async_copy(ei_hbm.at[wid, 1, u + 4], didx.at[u + 4], semi.at[u + 4])
        _scatter(u, u, u)

    def body(i, _):
        for u in range(8):
            cc = 4 + i * 8 + u
            q8 = (4 + u) % 8
            q4 = u % 4
            pltpu.make_async_copy(
                rows_v.at[0], acc_sh.at[didx.at[q8]], sems.at[q4]
            ).wait()
            pltpu.async_copy(
                ei_hbm.at[wid, 1, cc + 4], didx.at[(cc + 4) % 8], semi.at[(cc + 4) % 8]
            )
            _scatter(cc, q8, q4)
        return 0

    lax.fori_loop(0, (NCH - 8) // 8, body, 0)
    for u in range(4):
        cc = NCH - 4 + u
        q8 = cc % 8
        q4 = cc % 4
        pltpu.make_async_copy(
            rows_v.at[0], acc_sh.at[didx.at[q8]], sems.at[q4]
        ).wait()
        _scatter(cc, q8, q4)
    for q4 in range(4):
        pltpu.make_async_copy(
            rows_v.at[0], acc_sh.at[didx.at[0]], sems.at[q4]
        ).wait()

    plsc.subcore_barrier()
    _slab_writeout(acc_sh, out_hbm, c, base0, rows_v, sems)


@functools.partial(
    pl.kernel,
    out_type=jax.ShapeDtypeStruct((NC, NP, D), jnp.float32),
    mesh=_mesh,
    scratch_types=[
        pltpu.VMEM((2, K, D), jnp.float32),
        pltpu.VMEM((4, K), jnp.int32),
        pltpu.VMEM((4, K), jnp.int32),
        pltpu.VMEM_SHARED((NP, D), jnp.float32),
        pltpu.SemaphoreType.DMA((2,)),
        pltpu.SemaphoreType.DMA((2,)),
        pltpu.SemaphoreType.DMA((4,)),
        pltpu.SemaphoreType.DMA((4,)),
    ],
)
def _agg(g_hbm, ei_hbm, out_hbm, rows_v, sidx, didx, acc_sh, semg, sems, semis, semid):
    c = lax.axis_index("c")
    s = lax.axis_index("s")
    wid = s * NC + c
    base0 = s * RPT

    _zero_acc(acc_sh, base0, rows_v)
    plsc.subcore_barrier()

    def _load_idx(cc, q):
        pltpu.async_copy(ei_hbm.at[wid, 0, cc], sidx.at[q], semis.at[q])
        pltpu.async_copy(ei_hbm.at[wid, 1, cc], didx.at[q], semid.at[q])

    def _fire_gather(cc, q, p):
        pltpu.make_async_copy(
            ei_hbm.at[wid, 0, cc], sidx.at[q], semis.at[q]
        ).wait()
        pltpu.make_async_copy(
            ei_hbm.at[wid, 1, cc], didx.at[q], semid.at[q]
        ).wait()
        pltpu.async_copy(g_hbm.at[sidx.at[q]], rows_v.at[p], semg.at[p])

    _load_idx(0, 0)
    _load_idx(1, 1)
    _fire_gather(0, 0, 0)
    _fire_gather(1, 1, 1)
    pltpu.make_async_copy(g_hbm.at[sidx.at[0]], rows_v.at[0], semg.at[0]).wait()
    pltpu.async_copy(rows_v.at[0], acc_sh.at[didx.at[0]], sems.at[0], add=True)
    _load_idx(2, 2)
    pltpu.make_async_copy(rows_v.at[0], acc_sh.at[didx.at[0]], sems.at[0]).wait()
    _fire_gather(2, 2, 0)
    pltpu.make_async_copy(g_hbm.at[sidx.at[1]], rows_v.at[1], semg.at[1]).wait()
    pltpu.async_copy(rows_v.at[1], acc_sh.at[didx.at[1]], sems.at[1], add=True)
    _load_idx(3, 3)

    def body(i, _):
        for u in range(4):
            cc = 2 + i * 4 + u
            q = (2 + u) % 4
            p = u % 2
            pltpu.make_async_copy(
                rows_v.at[1 - p], acc_sh.at[didx.at[(q + 3) % 4]], sems.at[1 - p]
            ).wait()
            _fire_gather(cc + 1, (q + 1) % 4, 1 - p)
            pltpu.make_async_copy(
                g_hbm.at[sidx.at[q]], rows_v.at[p], semg.at[p]
            ).wait()
            pltpu.async_copy(
                rows_v.at[p], acc_sh.at[didx.at[q]], sems.at[p], add=True
            )
            _load_idx(cc + 2, (q + 2) % 4)
        return 0

    lax.fori_loop(0, (NCH - 4) // 4, body, 0)

    pltpu.make_async_copy(rows_v.at[1], acc_sh.at[didx.at[1]], sems.at[1]).wait()
    _fire_gather(79, 3, 1)
    pltpu.make_async_copy(g_hbm.at[sidx.at[2]], rows_v.at[0], semg.at[0]).wait()
    pltpu.async_copy(rows_v.at[0], acc_sh.at[didx.at[2]], sems.at[0], add=True)
    pltpu.make_async_copy(rows_v.at[0], acc_sh.at[didx.at[2]], sems.at[0]).wait()
    pltpu.make_async_copy(g_hbm.at[sidx.at[3]], rows_v.at[1], semg.at[1]).wait()
    pltpu.async_copy(rows_v.at[1], acc_sh.at[didx.at[3]], sems.at[1], add=True)
    pltpu.make_async_copy(rows_v.at[1], acc_sh.at[didx.at[3]], sems.at[1]).wait()

    plsc.subcore_barrier()
    _slab_writeout(acc_sh, out_hbm, c, base0, rows_v, sems)


def _tc1_body(x_ref, w_ref, degs_ref, g_ref, dinv_ref):
    deg = degs_ref[0][0:N, 0:1] + degs_ref[1][0:N, 0:1] + 1.0
    dinv = lax.rsqrt(deg)
    dinv_ref[...] = dinv
    g_ref[...] = (
        jnp.dot(x_ref[...], w_ref[...], preferred_element_type=jnp.float32) * dinv
    )


def _tc_mid_body(acc_ref, g_ref, dinv_ref, b_ref, gamma_ref, beta_ref, w_ref, out_ref):
    dinv = dinv_ref[...]
    sacc = (acc_ref[0][0:N] + acc_ref[1][0:N] + g_ref[...]) * dinv + b_ref[...]
    h = jnp.maximum(sacc, 0.0)
    mean = jnp.mean(h, axis=0, keepdims=True)
    var = jnp.mean((h - mean) ** 2, axis=0, keepdims=True)
    hn = (h - mean) * lax.rsqrt(var + EPS) * gamma_ref[...] + beta_ref[...]
    out_ref[...] = (
        jnp.dot(hn, w_ref[...], preferred_element_type=jnp.float32) * dinv
    )


def _tc_fin_body(acc_ref, g_ref, dinv_ref, b_ref, wc_ref, bc_ref, out_ref):
    sacc = (acc_ref[0][0:N] + acc_ref[1][0:N] + g_ref[...]) * dinv_ref[...] + b_ref[...]
    h = jnp.maximum(sacc, 0.0)
    pooled = jnp.mean(h, axis=0, keepdims=True)
    out_ref[...] = (
        jnp.dot(pooled, wc_ref[...], preferred_element_type=jnp.float32) + bc_ref[...]
    )


_tc1 = pl.pallas_call(
    _tc1_body,
    out_shape=(
        jax.ShapeDtypeStruct((N, D), jnp.float32),
        jax.ShapeDtypeStruct((N, 1), jnp.float32),
    ),
)

_tc_mid = pl.pallas_call(
    _tc_mid_body,
    out_shape=jax.ShapeDtypeStruct((N, D), jnp.float32),
)

_tc_fin = pl.pallas_call(
    _tc_fin_body,
    out_shape=jax.ShapeDtypeStruct((1, DOUT), jnp.float32),
)


def kernel(x, edge_index, W1, b1, W2, b2, W3, b3, gamma, beta, Wc, bc):
    npad = EP - E
    ei32 = edge_index.astype(jnp.int32)
    pad_src = (jnp.arange(npad, dtype=jnp.int32) * 131) % N
    pad_dst = N + (jnp.arange(npad, dtype=jnp.int32) % (NP - N))
    srcf = jnp.concatenate([ei32[0], pad_src])
    dstf = jnp.concatenate([ei32[1], pad_dst])
    ei = (
        jnp.stack([srcf, dstf])
        .reshape(2, NW, NCH, K)
        .transpose(1, 0, 2, 3)
    )
    b1r = b1.reshape(1, D)
    b2r = b2.reshape(1, D)
    b3r = b3.reshape(1, D)
    gr = gamma.reshape(1, D)
    br = beta.reshape(1, D)
    bcr = bc.reshape(1, DOUT)

    degs = _deg(ei)
    g1, dinv = _tc1(x, W1, degs)
    acc1 = _agg(g1, ei)
    g2 = _tc_mid(acc1, g1, dinv, b1r, gr, br, W2)
    acc2 = _agg(g2, ei)
    g3 = _tc_mid(acc2, g2, dinv, b2r, gr, br, W3)
    acc3 = _agg(g3, ei)
    return _tc_fin(acc3, g3, dinv, b3r, Wc, bcr)
